# Initial kernel scaffold; baseline (speedup 1.0000x reference)
import jax
import jax.numpy as jnp
from jax.experimental import pallas as pl


def kernel(features, rois):
    raise NotImplementedError("write your pallas kernel here")


# speedup vs baseline: 39.3021x; 39.3021x over previous
import functools

import jax
import jax.numpy as jnp
from jax import lax
from jax.experimental import pallas as pl
from jax.experimental.pallas import tpu as pltpu
from jax.experimental.pallas import tpu_sc as plsc

POOL = 7
LANES = 16
H = 224
W = 224
C = 96
CB = C // LANES
WBLK = 68
ROWW = WBLK * C
CH_ROWS = 8
N_WORKERS = 32
ROIS_PER_W = 32
OUT_W = POOL * POOL * C


def _bin_specs(rois, fh, fw):
    def fs(a, n):
        k = (n & -n).bit_length() - 1
        odd = n >> k
        bits = lax.bitcast_convert_type(a, jnp.uint32)
        e = (bits >> 23).astype(jnp.int32) - 127
        m = ((bits & 0x7FFFFF) | 0x800000).astype(jnp.int32)
        sh = jnp.clip(23 - k - e, 0, 31)
        return (odd * m) >> sh

    h_start = fs(rois[:, 0], fh)
    w_start = fs(rois[:, 1], fw)
    rh = fs(rois[:, 2], fh) - h_start
    rw = fs(rois[:, 3], fw) - w_start
    h_step = jnp.maximum(rh // POOL, 1)
    w_step = jnp.maximum(rw // POOL, 1)
    z = jnp.zeros_like(h_start)
    return jnp.stack(
        [h_start, w_start, h_step, w_step, rh, rw] + [z] * 10, axis=1
    )


def _sc_body(fm, specs, out, spec_v, rowbuf, acc):
    wid = lax.axis_index("s") * 2 + lax.axis_index("c")
    pltpu.sync_copy(specs.at[pl.ds(wid * (ROIS_PER_W * 16), ROIS_PER_W * 16)], spec_v)
    neg = jnp.full((LANES,), -jnp.inf, jnp.float32)

    def do_roi(t, carry):
        sv = spec_v[pl.ds(t * 16, 16)]
        h_start = sv[0]
        w_start = sv[1]
        h_step = sv[2]
        w_step = sv[3]
        rh = sv[4]
        rw = sv[5]
        woff = w_start * C

        def ini(i, c2):
            acc[pl.ds(i * LANES, LANES)] = neg
            return c2

        lax.fori_loop(0, OUT_W // LANES, ini, 0)

        nck = (rh + CH_ROWS - 1) // CH_ROWS
        last_w = rw - (POOL - 1) * w_step

        def do_chunk(ck, c2):
            r0 = h_start + ck * CH_ROWS
            pltpu.sync_copy(fm.at[pl.ds(r0, CH_ROWS), pl.ds(woff, ROWW)], rowbuf.at[0])
            nr = jnp.minimum(rh - ck * CH_ROWS, CH_ROWS)

            def do_row(rl, c3):
                r = ck * CH_ROWS + rl
                rbin = jnp.minimum(r // h_step, POOL - 1)
                abase = rbin * (POOL * C)
                for j in range(POOL):
                    ncj = w_step if j < POOL - 1 else last_w
                    base = j * w_step * C

                    def cb(c, car):
                        o = base + c * C
                        return tuple(
                            jnp.maximum(car[k], rowbuf[0, rl, pl.ds(o + LANES * k, LANES)])
                            for k in range(CB)
                        )

                    a = lax.fori_loop(0, ncj, cb, (neg,) * CB)
                    for k in range(CB):
                        sl = pl.ds(abase + j * C + LANES * k, LANES)
                        acc[sl] = jnp.maximum(acc[sl], a[k])
                return c3

            lax.fori_loop(0, nr, do_row, 0)
            return c2

        lax.fori_loop(0, nck, do_chunk, 0)
        pltpu.sync_copy(acc, out.at[wid * ROIS_PER_W + t])
        return carry

    lax.fori_loop(0, ROIS_PER_W, do_roi, 0)


def kernel(features, rois):
    n = rois.shape[0]
    fm = features.reshape(H, W * C)
    specs = _bin_specs(rois, H, W)
    n_pad = N_WORKERS * ROIS_PER_W
    pad_row = jnp.array([[0, 0, 1, 1] + [0] * 12], jnp.int32)
    specs = jnp.concatenate([specs, jnp.tile(pad_row, (n_pad - n, 1))], axis=0)
    specs = specs.reshape(-1)

    mesh = plsc.VectorSubcoreMesh(
        core_axis_name="c", subcore_axis_name="s", num_cores=2, num_subcores=16
    )
    run = pl.kernel(
        _sc_body,
        out_type=jax.ShapeDtypeStruct((n_pad, OUT_W), jnp.float32),
        mesh=mesh,
        compiler_params=pltpu.CompilerParams(use_tc_tiling_on_sc=False),
        scratch_types=[
            pltpu.VMEM((ROIS_PER_W * 16,), jnp.int32),
            pltpu.VMEM((1, CH_ROWS, ROWW), jnp.float32),
            pltpu.VMEM((OUT_W,), jnp.float32),
        ],
    )
    out = run(fm, specs)
    return out[:n].reshape(n, POOL, POOL, C)


# speedup vs baseline: 53.2583x; 1.3551x over previous
import functools

import jax
import jax.numpy as jnp
from jax import lax
from jax.experimental import pallas as pl
from jax.experimental.pallas import tpu as pltpu
from jax.experimental.pallas import tpu_sc as plsc

POOL = 7
LANES = 16
H = 224
W = 224
C = 96
CB = C // LANES
WBLK = 68
ROWW = WBLK * C
CH_ROWS = 8
N_WORKERS = 32
ROIS_PER_W = 32
OUT_W = POOL * POOL * C


def _bin_specs(rois, fh, fw):
    def fs(a, n):
        k = (n & -n).bit_length() - 1
        odd = n >> k
        bits = lax.bitcast_convert_type(a, jnp.uint32)
        e = (bits >> 23).astype(jnp.int32) - 127
        m = ((bits & 0x7FFFFF) | 0x800000).astype(jnp.int32)
        sh = jnp.clip(23 - k - e, 0, 31)
        return (odd * m) >> sh

    h_start = fs(rois[:, 0], fh)
    w_start = fs(rois[:, 1], fw)
    rh = fs(rois[:, 2], fh) - h_start
    rw = fs(rois[:, 3], fw) - w_start
    h_step = jnp.maximum(rh // POOL, 1)
    w_step = jnp.maximum(rw // POOL, 1)
    z = jnp.zeros_like(h_start)
    return jnp.stack(
        [h_start, w_start, h_step, w_step, rh, rw] + [z] * 10, axis=1
    )


def _sc_body(fm, specs, out, spec_v, rowbuf, acc, sem):
    wid = lax.axis_index("s") * 2 + lax.axis_index("c")
    pltpu.sync_copy(specs.at[pl.ds(wid * (ROIS_PER_W * 16), ROIS_PER_W * 16)], spec_v)
    neg = jnp.full((LANES,), -jnp.inf, jnp.float32)

    def do_roi(t, carry):
        sv = spec_v[pl.ds(t * 16, 16)]
        h_start = sv[0]
        w_start = sv[1]
        h_step = sv[2]
        w_step = sv[3]
        rh = sv[4]
        rw = sv[5]
        woff = w_start * C

        def ini(i, c2):
            acc[pl.ds(i * LANES, LANES)] = neg
            return c2

        lax.fori_loop(0, OUT_W // LANES, ini, 0)

        nck = (rh + CH_ROWS - 1) // CH_ROWS
        last_w = rw - (POOL - 1) * w_step

        def chunk_copy(ck, p):
            r0 = h_start + ck * CH_ROWS
            return pltpu.make_async_copy(
                fm.at[pl.ds(r0, CH_ROWS), pl.ds(woff, ROWW)], rowbuf.at[p], sem.at[p]
            )

        @pl.when(nck > 0)
        def _():
            chunk_copy(0, 0).start()

        def do_chunk(ck, c2):
            p = lax.rem(ck, 2)
            chunk_copy(ck, p).wait()

            @pl.when(ck + 1 < nck)
            def _():
                chunk_copy(ck + 1, 1 - p).start()

            nr = jnp.minimum(rh - ck * CH_ROWS, CH_ROWS)

            def do_row(rl, c3):
                r = ck * CH_ROWS + rl
                rbin = jnp.minimum(r // h_step, POOL - 1)
                abase = rbin * (POOL * C)
                for j in range(POOL):
                    ncj = w_step if j < POOL - 1 else last_w
                    base = j * w_step * C

                    def cb(c, car):
                        o = base + c * C
                        return tuple(
                            jnp.maximum(car[k], rowbuf[p, rl, pl.ds(o + LANES * k, LANES)])
                            for k in range(CB)
                        )

                    a = lax.fori_loop(0, ncj, cb, (neg,) * CB)
                    for k in range(CB):
                        sl = pl.ds(abase + j * C + LANES * k, LANES)
                        acc[sl] = jnp.maximum(acc[sl], a[k])
                return c3

            lax.fori_loop(0, nr, do_row, 0)
            return c2

        lax.fori_loop(0, nck, do_chunk, 0)
        pltpu.sync_copy(acc, out.at[wid * ROIS_PER_W + t])
        return carry

    lax.fori_loop(0, ROIS_PER_W, do_roi, 0)


def kernel(features, rois):
    n = rois.shape[0]
    fm = features.reshape(H, W * C)
    specs = _bin_specs(rois, H, W)
    n_pad = N_WORKERS * ROIS_PER_W
    pad_row = jnp.array([[0, 0, 1, 1] + [0] * 12], jnp.int32)
    specs = jnp.concatenate([specs, jnp.tile(pad_row, (n_pad - n, 1))], axis=0)
    specs = specs.reshape(-1)

    mesh = plsc.VectorSubcoreMesh(
        core_axis_name="c", subcore_axis_name="s", num_cores=2, num_subcores=16
    )
    run = pl.kernel(
        _sc_body,
        out_type=jax.ShapeDtypeStruct((n_pad, OUT_W), jnp.float32),
        mesh=mesh,
        compiler_params=pltpu.CompilerParams(use_tc_tiling_on_sc=False),
        scratch_types=[
            pltpu.VMEM((ROIS_PER_W * 16,), jnp.int32),
            pltpu.VMEM((2, CH_ROWS, ROWW), jnp.float32),
            pltpu.VMEM((OUT_W,), jnp.float32),
            pltpu.SemaphoreType.DMA((2,)),
        ],
    )
    out = run(fm, specs)
    return out[:n].reshape(n, POOL, POOL, C)


# speedup vs baseline: 55.3177x; 1.0387x over previous
import functools

import jax
import jax.numpy as jnp
from jax import lax
from jax.experimental import pallas as pl
from jax.experimental.pallas import tpu as pltpu
from jax.experimental.pallas import tpu_sc as plsc

POOL = 7
LANES = 16
H = 224
W = 224
C = 96
CB = C // LANES
WBLK = 68
ROWW = WBLK * C
CH_ROWS = 8
N_WORKERS = 32
ROIS_PER_W = 32
OUT_W = POOL * POOL * C


def _bin_specs(rois, fh, fw):
    def fs(a, n):
        k = (n & -n).bit_length() - 1
        odd = n >> k
        bits = lax.bitcast_convert_type(a, jnp.uint32)
        e = (bits >> 23).astype(jnp.int32) - 127
        m = ((bits & 0x7FFFFF) | 0x800000).astype(jnp.int32)
        sh = jnp.clip(23 - k - e, 0, 31)
        return (odd * m) >> sh

    h_start = fs(rois[:, 0], fh)
    w_start = fs(rois[:, 1], fw)
    rh = fs(rois[:, 2], fh) - h_start
    rw = fs(rois[:, 3], fw) - w_start
    h_step = jnp.maximum(rh // POOL, 1)
    w_step = jnp.maximum(rw // POOL, 1)
    z = jnp.zeros_like(h_start)
    return jnp.stack(
        [h_start, w_start, h_step, w_step, rh, rw] + [z] * 10, axis=1
    )


def _sc_body(fm, specs, out, spec_v, rowbuf, acc, sem):
    wid = lax.axis_index("s") * 2 + lax.axis_index("c")
    pltpu.sync_copy(specs.at[pl.ds(wid * (ROIS_PER_W * 16), ROIS_PER_W * 16)], spec_v)
    neg = jnp.full((LANES,), -jnp.inf, jnp.float32)

    def do_roi(t, carry):
        sv = spec_v[pl.ds(t * 16, 16)]
        h_start = sv[0]
        w_start = sv[1]
        h_step = sv[2]
        w_step = sv[3]
        rh = sv[4]
        rw = sv[5]
        woff = w_start * C

        @plsc.parallel_loop(0, OUT_W // LANES, unroll=7)
        def _(i):
            acc[pl.ds(i * LANES, LANES)] = neg

        nck = (rh + CH_ROWS - 1) // CH_ROWS
        last_w = rw - (POOL - 1) * w_step

        def chunk_copy(ck, p):
            r0 = h_start + ck * CH_ROWS
            return pltpu.make_async_copy(
                fm.at[pl.ds(r0, CH_ROWS), pl.ds(woff, ROWW)], rowbuf.at[p], sem.at[p]
            )

        @pl.when(nck > 0)
        def _():
            chunk_copy(0, 0).start()

        def do_chunk(ck, c2):
            p = lax.rem(ck, 2)
            chunk_copy(ck, p).wait()

            @pl.when(ck + 1 < nck)
            def _():
                chunk_copy(ck + 1, 1 - p).start()

            nr = jnp.minimum(rh - ck * CH_ROWS, CH_ROWS)

            def do_row(rl, c3):
                r = ck * CH_ROWS + rl
                rbin = jnp.minimum(r // h_step, POOL - 1)
                abase = rbin * (POOL * C)
                for j in range(POOL):
                    ncj = w_step if j < POOL - 1 else last_w
                    base = j * w_step * C

                    @plsc.parallel_loop(0, ncj, unroll=3, carry=(neg,) * CB)
                    def a(c, car):
                        o = base + c * C
                        return tuple(
                            jnp.maximum(car[k], rowbuf[p, rl, pl.ds(o + LANES * k, LANES)])
                            for k in range(CB)
                        )
                    for k in range(CB):
                        sl = pl.ds(abase + j * C + LANES * k, LANES)
                        acc[sl] = jnp.maximum(acc[sl], a[k])
                return c3

            lax.fori_loop(0, nr, do_row, 0)
            return c2

        lax.fori_loop(0, nck, do_chunk, 0)
        pltpu.sync_copy(acc, out.at[wid * ROIS_PER_W + t])
        return carry

    lax.fori_loop(0, ROIS_PER_W, do_roi, 0)


def kernel(features, rois):
    n = rois.shape[0]
    fm = features.reshape(H, W * C)
    specs = _bin_specs(rois, H, W)
    n_pad = N_WORKERS * ROIS_PER_W
    pad_row = jnp.array([[0, 0, 1, 1] + [0] * 12], jnp.int32)
    specs = jnp.concatenate([specs, jnp.tile(pad_row, (n_pad - n, 1))], axis=0)
    specs = specs.reshape(-1)

    mesh = plsc.VectorSubcoreMesh(
        core_axis_name="c", subcore_axis_name="s", num_cores=2, num_subcores=16
    )
    run = pl.kernel(
        _sc_body,
        out_type=jax.ShapeDtypeStruct((n_pad, OUT_W), jnp.float32),
        mesh=mesh,
        compiler_params=pltpu.CompilerParams(use_tc_tiling_on_sc=False),
        scratch_types=[
            pltpu.VMEM((ROIS_PER_W * 16,), jnp.int32),
            pltpu.VMEM((2, CH_ROWS, ROWW), jnp.float32),
            pltpu.VMEM((OUT_W,), jnp.float32),
            pltpu.SemaphoreType.DMA((2,)),
        ],
    )
    out = run(fm, specs)
    return out[:n].reshape(n, POOL, POOL, C)


# speedup vs baseline: 59.7467x; 1.0801x over previous
import functools

import jax
import jax.numpy as jnp
from jax import lax
from jax.experimental import pallas as pl
from jax.experimental.pallas import tpu as pltpu
from jax.experimental.pallas import tpu_sc as plsc

POOL = 7
LANES = 16
H = 224
W = 224
C = 96
CB = C // LANES
WBLK = 68
ROWW = WBLK * C
CH_ROWS = 8
N_WORKERS = 32
ROIS_PER_W = 32
OUT_W = POOL * POOL * C


def _bin_specs(rois, fh, fw):
    def fs(a, n):
        k = (n & -n).bit_length() - 1
        odd = n >> k
        bits = lax.bitcast_convert_type(a, jnp.uint32)
        e = (bits >> 23).astype(jnp.int32) - 127
        m = ((bits & 0x7FFFFF) | 0x800000).astype(jnp.int32)
        sh = jnp.clip(23 - k - e, 0, 31)
        return (odd * m) >> sh

    h_start = fs(rois[:, 0], fh)
    w_start = fs(rois[:, 1], fw)
    rh = fs(rois[:, 2], fh) - h_start
    rw = fs(rois[:, 3], fw) - w_start
    h_step = jnp.maximum(rh // POOL, 1)
    w_step = jnp.maximum(rw // POOL, 1)
    z = jnp.zeros_like(h_start)
    return jnp.stack(
        [h_start, w_start, h_step, w_step, rh, rw] + [z] * 10, axis=1
    )


def _sc_body(fm, specs, out, spec_v, rowbuf, acc, sem):
    wid = lax.axis_index("s") * 2 + lax.axis_index("c")
    pltpu.sync_copy(specs.at[pl.ds(wid * (ROIS_PER_W * 16), ROIS_PER_W * 16)], spec_v)
    neg = jnp.full((LANES,), -jnp.inf, jnp.float32)

    def do_roi(t, carry):
        sv = spec_v[pl.ds(t * 16, 16)]
        h_start = sv[0]
        w_start = sv[1]
        h_step = sv[2]
        w_step = sv[3]
        rh = sv[4]
        rw = sv[5]
        woff = w_start * C

        @plsc.parallel_loop(0, OUT_W // LANES, unroll=7)
        def _(i):
            acc[pl.ds(i * LANES, LANES)] = neg

        nck = (rh + CH_ROWS - 1) // CH_ROWS
        last_w = rw - (POOL - 1) * w_step

        def row_copy(ck, rl, p):
            r = h_start + ck * CH_ROWS + rl
            return pltpu.make_async_copy(
                fm.at[pl.ds(r * (W * C) + woff, ROWW)], rowbuf.at[p, rl], sem.at[p]
            )

        def start_chunk(ck, p):
            nr = jnp.minimum(rh - ck * CH_ROWS, CH_ROWS)

            def go(rl, c4):
                row_copy(ck, rl, p).start()
                return c4

            lax.fori_loop(0, nr, go, 0)

        def wait_chunk(ck, p):
            nr = jnp.minimum(rh - ck * CH_ROWS, CH_ROWS)

            def wt(rl, c4):
                row_copy(ck, rl, p).wait()
                return c4

            lax.fori_loop(0, nr, wt, 0)

        @pl.when(nck > 0)
        def _():
            start_chunk(0, 0)

        def do_chunk(ck, c2):
            p = lax.rem(ck, 2)
            wait_chunk(ck, p)

            @pl.when(ck + 1 < nck)
            def _():
                start_chunk(ck + 1, 1 - p)

            nr = jnp.minimum(rh - ck * CH_ROWS, CH_ROWS)

            def do_row(rl, c3):
                r = ck * CH_ROWS + rl
                rbin = jnp.minimum(r // h_step, POOL - 1)
                abase = rbin * (POOL * C)
                for j in range(POOL):
                    ncj = w_step if j < POOL - 1 else last_w
                    base = j * w_step * C

                    @plsc.parallel_loop(0, ncj, unroll=3, carry=(neg,) * CB)
                    def a(c, car):
                        o = base + c * C
                        return tuple(
                            jnp.maximum(car[k], rowbuf[p, rl, pl.ds(o + LANES * k, LANES)])
                            for k in range(CB)
                        )
                    for k in range(CB):
                        sl = pl.ds(abase + j * C + LANES * k, LANES)
                        acc[sl] = jnp.maximum(acc[sl], a[k])
                return c3

            lax.fori_loop(0, nr, do_row, 0)
            return c2

        lax.fori_loop(0, nck, do_chunk, 0)
        pltpu.sync_copy(acc, out.at[pl.ds((wid * ROIS_PER_W + t) * OUT_W, OUT_W)])
        return carry

    lax.fori_loop(0, ROIS_PER_W, do_roi, 0)


def kernel(features, rois):
    n = rois.shape[0]
    fm = features.reshape(H * W * C)
    specs = _bin_specs(rois, H, W)
    n_pad = N_WORKERS * ROIS_PER_W
    pad_row = jnp.array([[0, 0, 1, 1] + [0] * 12], jnp.int32)
    specs = jnp.concatenate([specs, jnp.tile(pad_row, (n_pad - n, 1))], axis=0)
    specs = specs.reshape(-1)

    mesh = plsc.VectorSubcoreMesh(
        core_axis_name="c", subcore_axis_name="s", num_cores=2, num_subcores=16
    )
    run = pl.kernel(
        _sc_body,
        out_type=jax.ShapeDtypeStruct((n_pad * OUT_W,), jnp.float32),
        mesh=mesh,
        compiler_params=pltpu.CompilerParams(use_tc_tiling_on_sc=False),
        scratch_types=[
            pltpu.VMEM((ROIS_PER_W * 16,), jnp.int32),
            pltpu.VMEM((2, CH_ROWS, ROWW), jnp.float32),
            pltpu.VMEM((OUT_W,), jnp.float32),
            pltpu.SemaphoreType.DMA((2,)),
        ],
    )
    out = run(fm, specs)
    return out[: n * OUT_W].reshape(n, POOL, POOL, C)


# speedup vs baseline: 59.7801x; 1.0006x over previous
import functools

import jax
import jax.numpy as jnp
from jax import lax
from jax.experimental import pallas as pl
from jax.experimental.pallas import tpu as pltpu
from jax.experimental.pallas import tpu_sc as plsc

POOL = 7
LANES = 16
H = 224
W = 224
C = 96
CB = C // LANES
WBLK = 68
ROWW = WBLK * C
CH_ROWS = 8
N_WORKERS = 32
ROIS_PER_W = 32
OUT_W = POOL * POOL * C


def _bin_specs(rois, fh, fw):
    def fs(a, n):
        k = (n & -n).bit_length() - 1
        odd = n >> k
        bits = lax.bitcast_convert_type(a, jnp.uint32)
        e = (bits >> 23).astype(jnp.int32) - 127
        m = ((bits & 0x7FFFFF) | 0x800000).astype(jnp.int32)
        sh = jnp.clip(23 - k - e, 0, 31)
        return (odd * m) >> sh

    h_start = fs(rois[:, 0], fh)
    w_start = fs(rois[:, 1], fw)
    rh = fs(rois[:, 2], fh) - h_start
    rw = fs(rois[:, 3], fw) - w_start
    h_step = jnp.maximum(rh // POOL, 1)
    w_step = jnp.maximum(rw // POOL, 1)
    z = jnp.zeros_like(h_start)
    return jnp.stack(
        [h_start, w_start, h_step, w_step, rh, rw] + [z] * 10, axis=1
    )


def _sc_body(fm, specs, out, spec_v, rowbuf, acc, sem):
    wid = lax.axis_index("s") * 2 + lax.axis_index("c")
    pltpu.sync_copy(
        specs.at[pl.ds(wid * (ROIS_PER_W * 16), (ROIS_PER_W + 1) * 16)], spec_v
    )
    neg = jnp.full((LANES,), -jnp.inf, jnp.float32)

    def get_spec(t):
        sv = spec_v[pl.ds(t * 16, 16)]
        return sv[0], sv[1], sv[2], sv[3], sv[4], sv[5]

    def row_copy(r, woff, rl, p):
        return pltpu.make_async_copy(
            fm.at[pl.ds(r * (W * C) + woff, ROWW)], rowbuf.at[p, rl], sem.at[p]
        )

    def start_chunk(hs, woff, rh, ck, p):
        nr = jnp.minimum(rh - ck * CH_ROWS, CH_ROWS)

        def go(rl, c4):
            row_copy(hs + ck * CH_ROWS + rl, woff, rl, p).start()
            return c4

        lax.fori_loop(0, nr, go, 0)

    def wait_chunk(hs, woff, rh, ck, p):
        nr = jnp.minimum(rh - ck * CH_ROWS, CH_ROWS)

        def wt(rl, c4):
            row_copy(hs + ck * CH_ROWS + rl, woff, rl, p).wait()
            return c4

        lax.fori_loop(0, nr, wt, 0)

    def out_copy(t, q):
        return pltpu.make_async_copy(
            acc.at[q],
            out.at[pl.ds((wid * ROIS_PER_W + t) * OUT_W, OUT_W)],
            sem.at[2 + q],
        )

    def do_roi(t, g):
        h_start, w_start, h_step, w_step, rh, rw = get_spec(t)
        woff = w_start * C
        q = lax.rem(t, 2)

        @pl.when(t >= 2)
        def _():
            out_copy(t - 2, q).wait()

        @plsc.parallel_loop(0, OUT_W // LANES, unroll=7)
        def _(i):
            acc[q, pl.ds(i * LANES, LANES)] = neg

        nck = (rh + CH_ROWS - 1) // CH_ROWS
        last_w = rw - (POOL - 1) * w_step

        @pl.when(nck > 0)
        def _():
            start_chunk(h_start, woff, rh, 0, 0)

        def do_chunk(ck, c2):
            p = lax.rem(ck, 2)
            wait_chunk(h_start, woff, rh, ck, p)

            @pl.when(ck + 1 < nck)
            def _():
                start_chunk(h_start, woff, rh, ck + 1, 1 - p)

            nr = jnp.minimum(rh - ck * CH_ROWS, CH_ROWS)

            def do_row(rl, c3):
                r = ck * CH_ROWS + rl
                rbin = jnp.minimum(r // h_step, POOL - 1)
                abase = rbin * (POOL * C)
                for j in range(POOL):
                    ncj = w_step if j < POOL - 1 else last_w
                    base = j * w_step * C

                    @plsc.parallel_loop(0, ncj, unroll=3, carry=(neg,) * CB)
                    def a(c, car):
                        o = base + c * C
                        return tuple(
                            jnp.maximum(car[k], rowbuf[p, rl, pl.ds(o + LANES * k, LANES)])
                            for k in range(CB)
                        )

                    for k in range(CB):
                        sl = pl.ds(abase + j * C + LANES * k, LANES)
                        acc[q, sl] = jnp.maximum(acc[q, sl], a[k])
                return c3

            lax.fori_loop(0, nr, do_row, 0)
            return c2

        lax.fori_loop(0, nck, do_chunk, 0)
        out_copy(t, q).start()
        return g + nck

    lax.fori_loop(0, ROIS_PER_W, do_roi, 0)
    out_copy(ROIS_PER_W - 2, 0).wait()
    out_copy(ROIS_PER_W - 1, 1).wait()


def kernel(features, rois):
    n = rois.shape[0]
    fm = features.reshape(H * W * C)
    specs = _bin_specs(rois, H, W)
    n_pad = N_WORKERS * ROIS_PER_W
    pad_row = jnp.array([[0, 0, 1, 1] + [0] * 12], jnp.int32)
    specs = jnp.concatenate(
        [specs, jnp.tile(pad_row, (n_pad + N_WORKERS - n, 1))], axis=0
    )
    specs = specs.reshape(-1)

    mesh = plsc.VectorSubcoreMesh(
        core_axis_name="c", subcore_axis_name="s", num_cores=2, num_subcores=16
    )
    run = pl.kernel(
        _sc_body,
        out_type=jax.ShapeDtypeStruct((n_pad * OUT_W,), jnp.float32),
        mesh=mesh,
        compiler_params=pltpu.CompilerParams(use_tc_tiling_on_sc=False),
        scratch_types=[
            pltpu.VMEM(((ROIS_PER_W + 1) * 16,), jnp.int32),
            pltpu.VMEM((2, CH_ROWS, ROWW), jnp.float32),
            pltpu.VMEM((2, OUT_W), jnp.float32),
            pltpu.SemaphoreType.DMA((4,)),
        ],
    )
    out = run(fm, specs)
    return out[: n * OUT_W].reshape(n, POOL, POOL, C)


# speedup vs baseline: 60.1640x; 1.0064x over previous
import functools

import jax
import jax.numpy as jnp
from jax import lax
from jax.experimental import pallas as pl
from jax.experimental.pallas import tpu as pltpu
from jax.experimental.pallas import tpu_sc as plsc

POOL = 7
LANES = 16
H = 224
W = 224
C = 96
CB = C // LANES
WBLK = 68
ROWW = WBLK * C
CH_ROWS = 8
N_WORKERS = 32
ROIS_PER_W = 32
OUT_W = POOL * POOL * C


def _bin_specs(rois, fh, fw):
    def fs(a, n):
        k = (n & -n).bit_length() - 1
        odd = n >> k
        bits = lax.bitcast_convert_type(a, jnp.uint32)
        e = (bits >> 23).astype(jnp.int32) - 127
        m = ((bits & 0x7FFFFF) | 0x800000).astype(jnp.int32)
        sh = jnp.clip(23 - k - e, 0, 31)
        return (odd * m) >> sh

    h_start = fs(rois[:, 0], fh)
    w_start = fs(rois[:, 1], fw)
    rh = fs(rois[:, 2], fh) - h_start
    rw = fs(rois[:, 3], fw) - w_start
    h_step = jnp.maximum(rh // POOL, 1)
    w_step = jnp.maximum(rw // POOL, 1)
    z = jnp.zeros_like(h_start)
    return jnp.stack(
        [h_start, w_start, h_step, w_step, rh, rw] + [z] * 10, axis=1
    )


def _sc_body(fm, specs, out, spec_v, rowbuf, acc, sem):
    wid = lax.axis_index("s") * 2 + lax.axis_index("c")
    pltpu.sync_copy(
        specs.at[pl.ds(wid * (ROIS_PER_W * 16), (ROIS_PER_W + 1) * 16)], spec_v
    )
    neg = jnp.full((LANES,), -jnp.inf, jnp.float32)

    def get_spec(t):
        sv = spec_v[pl.ds(t * 16, 16)]
        return sv[0], sv[1], sv[2], sv[3], sv[4], sv[5]

    def row_copy(r, woff, rl, p):
        return pltpu.make_async_copy(
            fm.at[pl.ds(r * (W * C) + woff, ROWW)], rowbuf.at[p, rl], sem.at[p]
        )

    def start_chunk(hs, woff, rh, ck, p):
        nr = jnp.minimum(rh - ck * CH_ROWS, CH_ROWS)

        def go(rl, c4):
            row_copy(hs + ck * CH_ROWS + rl, woff, rl, p).start()
            return c4

        lax.fori_loop(0, nr, go, 0)

    def wait_chunk(hs, woff, rh, ck, p):
        nr = jnp.minimum(rh - ck * CH_ROWS, CH_ROWS)

        def wt(rl, c4):
            row_copy(hs + ck * CH_ROWS + rl, woff, rl, p).wait()
            return c4

        lax.fori_loop(0, nr, wt, 0)

    def out_copy(t, q):
        return pltpu.make_async_copy(
            acc.at[q],
            out.at[pl.ds((wid * ROIS_PER_W + t) * OUT_W, OUT_W)],
            sem.at[2 + q],
        )

    hs0, ws0, _, _, rh0, _ = get_spec(0)

    @pl.when(rh0 > 0)
    def _():
        start_chunk(hs0, ws0 * C, rh0, 0, 0)

    def do_roi(t, g):
        h_start, w_start, h_step, w_step, rh, rw = get_spec(t)
        woff = w_start * C
        q = lax.rem(t, 2)

        @pl.when(t >= 2)
        def _():
            out_copy(t - 2, q).wait()

        @plsc.parallel_loop(0, OUT_W // LANES, unroll=7)
        def _(i):
            acc[q, pl.ds(i * LANES, LANES)] = neg

        nck = (rh + CH_ROWS - 1) // CH_ROWS
        last_w = rw - (POOL - 1) * w_step

        def do_chunk(ck, c2):
            p = lax.rem(g + ck, 2)
            wait_chunk(h_start, woff, rh, ck, p)

            @pl.when(ck + 1 < nck)
            def _():
                start_chunk(h_start, woff, rh, ck + 1, 1 - p)

            nr = jnp.minimum(rh - ck * CH_ROWS, CH_ROWS)

            def do_row(rl, c3):
                r = ck * CH_ROWS + rl
                rbin = jnp.minimum(r // h_step, POOL - 1)
                abase = rbin * (POOL * C)
                for j in range(POOL):
                    ncj = w_step if j < POOL - 1 else last_w
                    base = j * w_step * C

                    @plsc.parallel_loop(0, ncj, unroll=3, carry=(neg,) * CB)
                    def a(c, car):
                        o = base + c * C
                        return tuple(
                            jnp.maximum(car[k], rowbuf[p, rl, pl.ds(o + LANES * k, LANES)])
                            for k in range(CB)
                        )

                    for k in range(CB):
                        sl = pl.ds(abase + j * C + LANES * k, LANES)
                        acc[q, sl] = jnp.maximum(acc[q, sl], a[k])
                return c3

            lax.fori_loop(0, nr, do_row, 0)
            return c2

        lax.fori_loop(0, nck, do_chunk, 0)

        hs2, ws2, _, _, rh2, _ = get_spec(t + 1)

        @pl.when((t + 1 < ROIS_PER_W) & (rh2 > 0))
        def _():
            start_chunk(hs2, ws2 * C, rh2, 0, lax.rem(g + nck, 2))

        out_copy(t, q).start()
        return g + nck

    lax.fori_loop(0, ROIS_PER_W, do_roi, 0)
    out_copy(ROIS_PER_W - 2, 0).wait()
    out_copy(ROIS_PER_W - 1, 1).wait()


def kernel(features, rois):
    n = rois.shape[0]
    fm = features.reshape(H * W * C)
    specs = _bin_specs(rois, H, W)
    n_pad = N_WORKERS * ROIS_PER_W
    pad_row = jnp.array([[0, 0, 1, 1] + [0] * 12], jnp.int32)
    specs = jnp.concatenate(
        [specs, jnp.tile(pad_row, (n_pad + N_WORKERS - n, 1))], axis=0
    )
    specs = specs.reshape(-1)

    mesh = plsc.VectorSubcoreMesh(
        core_axis_name="c", subcore_axis_name="s", num_cores=2, num_subcores=16
    )
    run = pl.kernel(
        _sc_body,
        out_type=jax.ShapeDtypeStruct((n_pad * OUT_W,), jnp.float32),
        mesh=mesh,
        compiler_params=pltpu.CompilerParams(use_tc_tiling_on_sc=False),
        scratch_types=[
            pltpu.VMEM(((ROIS_PER_W + 1) * 16,), jnp.int32),
            pltpu.VMEM((2, CH_ROWS, ROWW), jnp.float32),
            pltpu.VMEM((2, OUT_W), jnp.float32),
            pltpu.SemaphoreType.DMA((4,)),
        ],
    )
    out = run(fm, specs)
    return out[: n * OUT_W].reshape(n, POOL, POOL, C)


# speedup vs baseline: 66.7215x; 1.1090x over previous
import functools

import jax
import jax.numpy as jnp
from jax import lax
from jax.experimental import pallas as pl
from jax.experimental.pallas import tpu as pltpu
from jax.experimental.pallas import tpu_sc as plsc

POOL = 7
LANES = 16
H = 224
W = 224
C = 96
CB = C // LANES
WBLK = 68
ROWW = WBLK * C
CH_ROWS = 8
N_WORKERS = 32
ROIS_PER_W = 32
OUT_W = POOL * POOL * C


def _bin_specs(rois, fh, fw):
    def fs(a, n):
        k = (n & -n).bit_length() - 1
        odd = n >> k
        bits = lax.bitcast_convert_type(a, jnp.uint32)
        e = (bits >> 23).astype(jnp.int32) - 127
        m = ((bits & 0x7FFFFF) | 0x800000).astype(jnp.int32)
        sh = jnp.clip(23 - k - e, 0, 31)
        return (odd * m) >> sh

    h_start = fs(rois[:, 0], fh)
    w_start = fs(rois[:, 1], fw)
    rh = fs(rois[:, 2], fh) - h_start
    rw = fs(rois[:, 3], fw) - w_start
    h_step = jnp.maximum(rh // POOL, 1)
    w_step = jnp.maximum(rw // POOL, 1)
    z = jnp.zeros_like(h_start)
    return jnp.stack(
        [h_start, w_start, h_step, w_step, rh, rw] + [z] * 10, axis=1
    )


def _sc_body(fm, specs, out, spec_v, rowbuf, acc, sem):
    wid = lax.axis_index("s") * 2 + lax.axis_index("c")
    pltpu.sync_copy(
        specs.at[pl.ds(wid * (ROIS_PER_W * 16), (ROIS_PER_W + 1) * 16)], spec_v
    )
    neg = jnp.full((LANES,), -jnp.inf, jnp.float32)

    def get_spec(t):
        sv = spec_v[pl.ds(t * 16, 16)]
        return sv[0], sv[1], sv[2], sv[3], sv[4], sv[5]

    def row_copy(r, woff, rl, p, nw):
        return pltpu.make_async_copy(
            fm.at[pl.ds(r * (W * C) + woff, nw)],
            rowbuf.at[p, rl, pl.ds(0, nw)],
            sem.at[p],
        )

    def chunk_io(hs, woff, rh, wcls, ck, p, do_start):
        nr = jnp.minimum(rh - ck * CH_ROWS, CH_ROWS)
        for i, nw in enumerate((32 * C, 48 * C, ROWW)):

            @pl.when(wcls == i)
            def _(nw=nw):
                def go(rl, c4):
                    d = row_copy(hs + ck * CH_ROWS + rl, woff, rl, p, nw)
                    if do_start:
                        d.start()
                    else:
                        d.wait()
                    return c4

                lax.fori_loop(0, nr, go, 0)

    def start_chunk(hs, woff, rh, wcls, ck, p):
        chunk_io(hs, woff, rh, wcls, ck, p, True)

    def wait_chunk(hs, woff, rh, wcls, ck, p):
        chunk_io(hs, woff, rh, wcls, ck, p, False)

    def wclass(rw):
        return (rw > 32).astype(jnp.int32) + (rw > 48).astype(jnp.int32)

    def out_copy(t, q):
        return pltpu.make_async_copy(
            acc.at[q],
            out.at[pl.ds((wid * ROIS_PER_W + t) * OUT_W, OUT_W)],
            sem.at[2 + q],
        )

    hs0, ws0, _, _, rh0, rw0 = get_spec(0)

    @pl.when(rh0 > 0)
    def _():
        start_chunk(hs0, ws0 * C, rh0, wclass(rw0), 0, 0)

    def do_roi(t, g):
        h_start, w_start, h_step, w_step, rh, rw = get_spec(t)
        woff = w_start * C
        q = lax.rem(t, 2)

        @pl.when(t >= 2)
        def _():
            out_copy(t - 2, q).wait()

        @plsc.parallel_loop(0, OUT_W // LANES, unroll=7)
        def _(i):
            acc[q, pl.ds(i * LANES, LANES)] = neg

        nck = (rh + CH_ROWS - 1) // CH_ROWS
        last_w = rw - (POOL - 1) * w_step

        wcls = wclass(rw)

        def do_chunk(ck, c2):
            p = lax.rem(g + ck, 2)
            wait_chunk(h_start, woff, rh, wcls, ck, p)

            @pl.when(ck + 1 < nck)
            def _():
                start_chunk(h_start, woff, rh, wcls, ck + 1, 1 - p)

            nr = jnp.minimum(rh - ck * CH_ROWS, CH_ROWS)

            def do_row(rl, c3):
                r = ck * CH_ROWS + rl
                rbin = jnp.minimum(r // h_step, POOL - 1)
                abase = rbin * (POOL * C)
                for j in range(POOL):
                    ncj = w_step if j < POOL - 1 else last_w
                    base = j * w_step * C

                    @plsc.parallel_loop(0, ncj, unroll=3, carry=(neg,) * CB)
                    def a(c, car):
                        o = base + c * C
                        return tuple(
                            jnp.maximum(car[k], rowbuf[p, rl, pl.ds(o + LANES * k, LANES)])
                            for k in range(CB)
                        )

                    for k in range(CB):
                        sl = pl.ds(abase + j * C + LANES * k, LANES)
                        acc[q, sl] = jnp.maximum(acc[q, sl], a[k])
                return c3

            lax.fori_loop(0, nr, do_row, 0)
            return c2

        lax.fori_loop(0, nck, do_chunk, 0)

        hs2, ws2, _, _, rh2, rw2 = get_spec(t + 1)

        @pl.when((t + 1 < ROIS_PER_W) & (rh2 > 0))
        def _():
            start_chunk(hs2, ws2 * C, rh2, wclass(rw2), 0, lax.rem(g + nck, 2))

        out_copy(t, q).start()
        return g + nck

    lax.fori_loop(0, ROIS_PER_W, do_roi, 0)
    out_copy(ROIS_PER_W - 2, 0).wait()
    out_copy(ROIS_PER_W - 1, 1).wait()


def kernel(features, rois):
    n = rois.shape[0]
    fm = features.reshape(H * W * C)
    specs = _bin_specs(rois, H, W)
    n_pad = N_WORKERS * ROIS_PER_W
    pad_row = jnp.array([[0, 0, 1, 1] + [0] * 12], jnp.int32)
    specs = jnp.concatenate(
        [specs, jnp.tile(pad_row, (n_pad + N_WORKERS - n, 1))], axis=0
    )
    specs = specs.reshape(-1)

    mesh = plsc.VectorSubcoreMesh(
        core_axis_name="c", subcore_axis_name="s", num_cores=2, num_subcores=16
    )
    run = pl.kernel(
        _sc_body,
        out_type=jax.ShapeDtypeStruct((n_pad * OUT_W,), jnp.float32),
        mesh=mesh,
        compiler_params=pltpu.CompilerParams(use_tc_tiling_on_sc=False),
        scratch_types=[
            pltpu.VMEM(((ROIS_PER_W + 1) * 16,), jnp.int32),
            pltpu.VMEM((2, CH_ROWS, ROWW), jnp.float32),
            pltpu.VMEM((2, OUT_W), jnp.float32),
            pltpu.SemaphoreType.DMA((4,)),
        ],
    )
    out = run(fm, specs)
    return out[: n * OUT_W].reshape(n, POOL, POOL, C)


# speedup vs baseline: 67.2243x; 1.0075x over previous
import functools

import jax
import jax.numpy as jnp
from jax import lax
from jax.experimental import pallas as pl
from jax.experimental.pallas import tpu as pltpu
from jax.experimental.pallas import tpu_sc as plsc

POOL = 7
LANES = 16
H = 224
W = 224
C = 96
CB = C // LANES
WBLK = 68
ROWW = WBLK * C
CH_ROWS = 8
N_WORKERS = 32
ROIS_PER_W = 32
OUT_W = POOL * POOL * C


def _bin_specs(rois, fh, fw):
    def fs(a, n):
        k = (n & -n).bit_length() - 1
        odd = n >> k
        bits = lax.bitcast_convert_type(a, jnp.uint32)
        e = (bits >> 23).astype(jnp.int32) - 127
        m = ((bits & 0x7FFFFF) | 0x800000).astype(jnp.int32)
        sh = jnp.clip(23 - k - e, 0, 31)
        return (odd * m) >> sh

    h_start = fs(rois[:, 0], fh)
    w_start = fs(rois[:, 1], fw)
    rh = fs(rois[:, 2], fh) - h_start
    rw = fs(rois[:, 3], fw) - w_start
    h_step = jnp.maximum(rh // POOL, 1)
    w_step = jnp.maximum(rw // POOL, 1)
    z = jnp.zeros_like(h_start)
    return jnp.stack(
        [h_start, w_start, h_step, w_step, rh, rw] + [z] * 10, axis=1
    )


def _sc_body(fm, specs, out, spec_v, rowbuf, acc, sem):
    wid = lax.axis_index("s") * 2 + lax.axis_index("c")
    pltpu.sync_copy(
        specs.at[pl.ds(wid * (ROIS_PER_W * 16), (ROIS_PER_W + 1) * 16)], spec_v
    )
    neg = jnp.full((LANES,), -jnp.inf, jnp.float32)

    def get_spec(t):
        sv = spec_v[pl.ds(t * 16, 16)]
        return sv[0], sv[1], sv[2], sv[3], sv[4], sv[5]

    def row_copy(r, woff, rl, p, nw):
        return pltpu.make_async_copy(
            fm.at[pl.ds(r * (W * C) + woff, nw)],
            rowbuf.at[p, rl, pl.ds(0, nw)],
            sem.at[p],
        )

    def chunk_io(hs, woff, rh, wcls, ck, p, do_start):
        nr = jnp.minimum(rh - ck * CH_ROWS, CH_ROWS)
        for i, nw in enumerate((24 * C, 32 * C, 40 * C, 48 * C, 56 * C, ROWW)):

            @pl.when(wcls == i)
            def _(nw=nw):
                def go(rl, c4):
                    d = row_copy(hs + ck * CH_ROWS + rl, woff, rl, p, nw)
                    if do_start:
                        d.start()
                    else:
                        d.wait()
                    return c4

                lax.fori_loop(0, nr, go, 0)

    def start_chunk(hs, woff, rh, wcls, ck, p):
        chunk_io(hs, woff, rh, wcls, ck, p, True)

    def wait_chunk(hs, woff, rh, wcls, ck, p):
        chunk_io(hs, woff, rh, wcls, ck, p, False)

    def wclass(rw):
        c = (rw > 24).astype(jnp.int32)
        for b in (32, 40, 48, 56):
            c = c + (rw > b).astype(jnp.int32)
        return c

    def out_copy(t, q):
        return pltpu.make_async_copy(
            acc.at[q],
            out.at[pl.ds((wid * ROIS_PER_W + t) * OUT_W, OUT_W)],
            sem.at[2 + q],
        )

    hs0, ws0, _, _, rh0, rw0 = get_spec(0)

    @pl.when(rh0 > 0)
    def _():
        start_chunk(hs0, ws0 * C, rh0, wclass(rw0), 0, 0)

    def do_roi(t, g):
        h_start, w_start, h_step, w_step, rh, rw = get_spec(t)
        woff = w_start * C
        q = lax.rem(t, 2)

        @pl.when(t >= 2)
        def _():
            out_copy(t - 2, q).wait()

        @plsc.parallel_loop(0, OUT_W // LANES, unroll=7)
        def _(i):
            acc[q, pl.ds(i * LANES, LANES)] = neg

        nck = (rh + CH_ROWS - 1) // CH_ROWS
        last_w = rw - (POOL - 1) * w_step

        wcls = wclass(rw)

        def do_chunk(ck, c2):
            p = lax.rem(g + ck, 2)
            wait_chunk(h_start, woff, rh, wcls, ck, p)

            @pl.when(ck + 1 < nck)
            def _():
                start_chunk(h_start, woff, rh, wcls, ck + 1, 1 - p)

            nr = jnp.minimum(rh - ck * CH_ROWS, CH_ROWS)

            def do_row(rl, c3):
                r = ck * CH_ROWS + rl
                rbin = jnp.minimum(r // h_step, POOL - 1)
                abase = rbin * (POOL * C)
                for j in range(POOL):
                    ncj = w_step if j < POOL - 1 else last_w
                    base = j * w_step * C

                    @plsc.parallel_loop(0, ncj, unroll=3, carry=(neg,) * CB)
                    def a(c, car):
                        o = base + c * C
                        return tuple(
                            jnp.maximum(car[k], rowbuf[p, rl, pl.ds(o + LANES * k, LANES)])
                            for k in range(CB)
                        )

                    for k in range(CB):
                        sl = pl.ds(abase + j * C + LANES * k, LANES)
                        acc[q, sl] = jnp.maximum(acc[q, sl], a[k])
                return c3

            lax.fori_loop(0, nr, do_row, 0)
            return c2

        lax.fori_loop(0, nck, do_chunk, 0)

        hs2, ws2, _, _, rh2, rw2 = get_spec(t + 1)

        @pl.when((t + 1 < ROIS_PER_W) & (rh2 > 0))
        def _():
            start_chunk(hs2, ws2 * C, rh2, wclass(rw2), 0, lax.rem(g + nck, 2))

        out_copy(t, q).start()
        return g + nck

    lax.fori_loop(0, ROIS_PER_W, do_roi, 0)
    out_copy(ROIS_PER_W - 2, 0).wait()
    out_copy(ROIS_PER_W - 1, 1).wait()


def kernel(features, rois):
    n = rois.shape[0]
    fm = features.reshape(H * W * C)
    specs = _bin_specs(rois, H, W)
    n_pad = N_WORKERS * ROIS_PER_W
    pad_row = jnp.array([[0, 0, 1, 1] + [0] * 12], jnp.int32)
    specs = jnp.concatenate(
        [specs, jnp.tile(pad_row, (n_pad + N_WORKERS - n, 1))], axis=0
    )
    specs = specs.reshape(-1)

    mesh = plsc.VectorSubcoreMesh(
        core_axis_name="c", subcore_axis_name="s", num_cores=2, num_subcores=16
    )
    run = pl.kernel(
        _sc_body,
        out_type=jax.ShapeDtypeStruct((n_pad * OUT_W,), jnp.float32),
        mesh=mesh,
        compiler_params=pltpu.CompilerParams(use_tc_tiling_on_sc=False),
        scratch_types=[
            pltpu.VMEM(((ROIS_PER_W + 1) * 16,), jnp.int32),
            pltpu.VMEM((2, CH_ROWS, ROWW), jnp.float32),
            pltpu.VMEM((2, OUT_W), jnp.float32),
            pltpu.SemaphoreType.DMA((4,)),
        ],
    )
    out = run(fm, specs)
    return out[: n * OUT_W].reshape(n, POOL, POOL, C)
